# baseline (speedup 1.0000x reference)
import functools
import jax
import jax.numpy as jnp
from jax import lax
from jax.experimental import pallas as pl
from jax.experimental.pallas import tpu as pltpu
from jax.experimental.pallas import tpu_sc as plsc

NUM_SEGMENTS = 16
D_MODEL = 4096

_info = plsc.get_sparse_core_info()
_NC, _NS = _info.num_cores, _info.num_subcores
_NW = _NC * _NS
_L = 16

_B = 4 * 8192
_BPW = _B // _NW
_G = _BPW // _L
_GPAIR = _G // 2


def _body(ids_hbm, table_hbm, out_hbm, idx_v, tab_v, sem0, sem1):
    wid = lax.axis_index("s") * _NC + lax.axis_index("c")
    base = wid * _BPW
    pltpu.sync_copy(ids_hbm.at[wid], idx_v)
    pltpu.sync_copy(table_hbm, tab_v)
    lanes = lax.iota(jnp.int32, _L)

    def issue_group(g, sem):
        v = idx_v[pl.ds(g * _L, _L)]
        for l in range(_L):
            s = jnp.sum(jnp.where(lanes == l, v, 0))
            pltpu.make_async_copy(
                tab_v.at[pl.ds(s, 1)],
                out_hbm.at[pl.ds(base + g * _L + l, 1)],
                sem,
            ).start()

    def drain_group(sem):
        d = pltpu.make_async_copy(
            tab_v.at[pl.ds(0, 1)], out_hbm.at[pl.ds(base, 1)], sem
        )
        for _ in range(_L):
            d.wait()

    def pair(gg, carry):
        @pl.when(gg > 1)
        def _():
            drain_group(sem0)
        issue_group(2 * gg, sem0)
        @pl.when(gg > 1)
        def _():
            drain_group(sem1)
        issue_group(2 * gg + 1, sem1)
        return carry

    lax.fori_loop(0, _GPAIR, pair, 0)
    drain_group(sem0)
    drain_group(sem1)
    drain_group(sem0)
    drain_group(sem1)


def kernel(segment_ids, table):
    ids = segment_ids.reshape(_NW, _BPW).astype(jnp.int32)
    run = functools.partial(
        pl.kernel,
        mesh=plsc.VectorSubcoreMesh(core_axis_name="c", subcore_axis_name="s"),
        out_type=jax.ShapeDtypeStruct((_B, D_MODEL), jnp.float32),
        compiler_params=pltpu.CompilerParams(needs_layout_passes=False),
        scratch_types=[
            pltpu.VMEM((_BPW,), jnp.int32),
            pltpu.VMEM((NUM_SEGMENTS, D_MODEL), jnp.float32),
            pltpu.SemaphoreType.DMA,
            pltpu.SemaphoreType.DMA,
        ],
    )(_body)
    out = run(ids, table)
    return out.reshape(segment_ids.shape[0], segment_ids.shape[1], D_MODEL)


# speedup vs baseline: 1.0328x; 1.0328x over previous
import functools
import jax
import jax.numpy as jnp
from jax import lax
from jax.experimental import pallas as pl
from jax.experimental.pallas import tpu as pltpu
from jax.experimental.pallas import tpu_sc as plsc

NUM_SEGMENTS = 16
D_MODEL = 4096

_info = plsc.get_sparse_core_info()
_NC, _NS = _info.num_cores, _info.num_subcores
_NW = _NC * _NS
_L = 16

_B = 4 * 8192
_BPW = _B // _NW
_G = _BPW // _L


def _body(ids_hbm, table_hbm, out_hbm, idx_v, tab_v, buckets, count_v, sem0, sem1):
    wid = lax.axis_index("s") * _NC + lax.axis_index("c")
    base = wid * _BPW
    pltpu.sync_copy(ids_hbm.at[wid], idx_v)
    pltpu.sync_copy(table_hbm, tab_v)
    lanes = lax.iota(jnp.int32, _L)
    count_v[...] = lanes * 0

    def bucketize(g, carry):
        v = idx_v[pl.ds(g * _L, _L)]
        p = base + g * _L + lanes
        sv, sp = plsc.sort_key_val(v, p)
        prev = sv.at[jnp.maximum(lanes - 1, 0)].get(mode="promise_in_bounds")
        start = (lanes == 0) | (sv != prev)
        run_start = plsc.cummax(jnp.where(start, lanes, 0))
        rank = lanes - run_start
        cb = plsc.load_gather(count_v, [sv])
        slot = cb + rank
        plsc.store_scatter(buckets, [sv, slot], sp)
        nxt = sv.at[jnp.minimum(lanes + 1, _L - 1)].get(mode="promise_in_bounds")
        last = (lanes == _L - 1) | (sv != nxt)
        plsc.store_scatter(count_v, [sv], slot + 1, mask=last)
        return carry

    lax.fori_loop(0, _G, bucketize, 0)

    cnt = count_v[...]
    mc = jnp.min(cnt)

    def seg_count(s):
        return jnp.sum(jnp.where(lanes == s, cnt, 0))

    def drain_a():
        pltpu.make_async_copy(tab_v, out_hbm.at[pl.ds(base, _L)], sem0).wait()

    def round_a(r, carry):
        @pl.when(r >= 2)
        def _():
            drain_a()
        posv = plsc.load_gather(buckets, [lanes, lanes * 0 + r])
        pltpu.make_async_copy(tab_v, out_hbm.at[posv], sem0).start()
        return carry

    lax.fori_loop(0, mc, round_a, 0)
    lax.fori_loop(0, jnp.minimum(mc, 2), lambda i, c: (drain_a(), c)[1], 0)

    def row_copy(s, sem):
        def row(r, carry):
            pv = plsc.load_gather(buckets, [lanes * 0 + s, lanes * 0 + r])
            pos = jnp.max(pv)
            pltpu.make_async_copy(
                tab_v.at[pl.ds(s, 1)], out_hbm.at[pl.ds(pos, 1)], sem
            ).start()
            return carry
        lax.fori_loop(mc, seg_count(s), row, 0)

    def drain_b(n, sem):
        d = pltpu.make_async_copy(
            tab_v.at[pl.ds(0, 1)], out_hbm.at[pl.ds(base, 1)], sem
        )
        lax.fori_loop(0, n, lambda i, c: (d.wait(), c)[1], 0)

    sems = (sem0, sem1)
    for s in range(NUM_SEGMENTS):
        row_copy(s, sems[s % 2])
        if s >= 1:
            drain_b(seg_count(s - 1) - mc, sems[(s - 1) % 2])
    drain_b(seg_count(NUM_SEGMENTS - 1) - mc, sems[(NUM_SEGMENTS - 1) % 2])


def kernel(segment_ids, table):
    ids = segment_ids.reshape(_NW, _BPW).astype(jnp.int32)
    run = functools.partial(
        pl.kernel,
        mesh=plsc.VectorSubcoreMesh(core_axis_name="c", subcore_axis_name="s"),
        out_type=jax.ShapeDtypeStruct((_B, D_MODEL), jnp.float32),
        compiler_params=pltpu.CompilerParams(needs_layout_passes=False),
        scratch_types=[
            pltpu.VMEM((_BPW,), jnp.int32),
            pltpu.VMEM((NUM_SEGMENTS, D_MODEL), jnp.float32),
            pltpu.VMEM((NUM_SEGMENTS, _BPW), jnp.int32),
            pltpu.VMEM((_L,), jnp.int32),
            pltpu.SemaphoreType.DMA,
            pltpu.SemaphoreType.DMA,
        ],
    )(_body)
    out = run(ids, table)
    return out.reshape(segment_ids.shape[0], segment_ids.shape[1], D_MODEL)
